# baseline (speedup 1.0000x reference)
import functools
import math

import jax
import jax.numpy as jnp
from jax import lax
from jax.experimental import pallas as pl
from jax.experimental.pallas import tpu as pltpu
from jax.experimental.pallas import tpu_sc as plsc

D_MODEL = 64
DPAD = 128
SCALE = math.sqrt(D_MODEL)

NC, NS, L = 2, 16, 16
NW = NC * NS

NBUF = 4
SUB = ((0, 104), (104, 96))


def _make_format_kernel(V):
    full_chunks = V // DPAD
    tail_cols = V - full_chunks * DPAD
    iters = ((full_chunks + NW - 1) // NW + 1) // 2 * 2
    mesh = plsc.VectorSubcoreMesh(core_axis_name="c", subcore_axis_name="s")

    scratch = [pltpu.VMEM((D_MODEL, DPAD), jnp.float32) for _ in range(4)]
    scratch += [pltpu.SemaphoreType.DMA for _ in range(4)]

    @functools.partial(
        pl.kernel,
        out_type=jax.ShapeDtypeStruct((V // 2, DPAD), jnp.float32),
        mesh=mesh,
        scratch_types=scratch,
        compiler_params=pltpu.CompilerParams(
            use_tc_tiling_on_sc=True, needs_layout_passes=False
        ),
    )
    def fmt_kernel(wt_hbm, out_hbm, *bufs_and_sems):
        ins = bufs_and_sems[0:2]
        outs = bufs_and_sems[2:4]
        rsem = bufs_and_sems[4:6]
        wsem = bufs_and_sems[6:8]

        wid = lax.axis_index("s") * NC + lax.axis_index("c")
        lanes = lax.iota(jnp.int32, L)

        def chunk_of(i):
            c = wid + i * NW
            return jnp.minimum(c, full_chunks - 1)

        def fire_read(i, b):
            c = chunk_of(i)
            pltpu.async_copy(
                wt_hbm.at[pl.ds(0, D_MODEL), pl.ds(c * DPAD, DPAD)],
                ins[b],
                rsem[b],
            )

        def wait_read(b):
            pltpu.make_async_copy(
                wt_hbm.at[pl.ds(0, D_MODEL), pl.ds(0, DPAD)], ins[b], rsem[b]
            ).wait()

        def wait_write(b):
            pltpu.make_async_copy(outs[b], out_hbm.at[pl.ds(0, D_MODEL)], wsem[b]).wait()

        rows16s = [lanes + (16 * k) % D_MODEL for k in range(DPAD // L)]
        zeros16 = jnp.full((L,), 0, jnp.int32)

        def transpose_scale(b, nrows):
            src = ins[b]
            dst = outs[b]

            @plsc.parallel_loop(0, nrows, unroll=16)
            def _(j):
                c0 = zeros16 + 2 * j
                c1 = c0 + 1
                for k in range(DPAD // L):
                    cols16 = c0 if k < D_MODEL // L else c1
                    v = plsc.load_gather(src, [rows16s[k], cols16])
                    dst[j, pl.ds(k * L, L)] = v * SCALE

        fire_read(0, 0)

        def body(i0, carry):
            for b in range(2):
                i = i0 * 2 + b
                nb = 1 - b

                @pl.when(i + 1 < iters)
                def _():
                    fire_read(i + 1, nb)

                wait_read(b)

                @pl.when(i >= 2)
                def _():
                    wait_write(b)

                transpose_scale(b, D_MODEL)
                c = chunk_of(i)
                pltpu.async_copy(
                    outs[b], out_hbm.at[pl.ds(c * D_MODEL, D_MODEL)], wsem[b]
                )
            return carry

        lax.fori_loop(0, iters // 2, body, 0)
        wait_write(0)
        wait_write(1)

    return fmt_kernel


def _make_lookup_kernel(B0, S, V):
    seq_per_w = B0 // NW
    G = seq_per_w * len(SUB)
    assert G % NBUF == 0
    cmax = max(c for _, c in SUB)
    mesh = plsc.VectorSubcoreMesh(core_axis_name="c", subcore_axis_name="s")

    scratch = [pltpu.VMEM((seq_per_w, S), jnp.int32)]
    scratch += [pltpu.VMEM((cmax, D_MODEL), jnp.float32) for _ in range(NBUF)]
    scratch += [pltpu.SemaphoreType.DMA for _ in range(2 * NBUF)]

    @functools.partial(
        pl.kernel,
        out_type=jax.ShapeDtypeStruct((B0, S, DPAD), jnp.float32),
        mesh=mesh,
        scratch_types=scratch,
        compiler_params=pltpu.CompilerParams(use_tc_tiling_on_sc=False),
    )
    def emb_kernel(x_hbm, w_hbm, out_hbm, idx_v, *bufs_and_sems):
        rows = bufs_and_sems[:NBUF]
        gsem = bufs_and_sems[NBUF:2 * NBUF]
        ssem = bufs_and_sems[2 * NBUF:]

        wid = lax.axis_index("s") * NC + lax.axis_index("c")
        s0 = wid * seq_per_w

        pltpu.sync_copy(x_hbm.at[pl.ds(s0, seq_per_w)], idx_v)

        def fire_gather(lseq, h, b):
            off, cnt = SUB[h]
            pltpu.async_copy(
                w_hbm.at[idx_v.at[lseq, pl.ds(off, cnt)]],
                rows[b].at[pl.ds(0, cnt)],
                gsem[b],
            )

        def drain_gather(h, b):
            off, cnt = SUB[h]
            pltpu.make_async_copy(
                w_hbm.at[idx_v.at[0, pl.ds(off, cnt)]],
                rows[b].at[pl.ds(0, cnt)],
                gsem[b],
            ).wait()

        def drain_scatter(h, b):
            off, cnt = SUB[h]
            pltpu.make_async_copy(
                rows[b].at[pl.ds(0, cnt)],
                out_hbm.at[0, pl.ds(off, cnt), pl.ds(0, D_MODEL)],
                ssem[b],
            ).wait()

        fire_gather(0, 0, 0)

        def outer(g0, carry):
            for b in range(NBUF):
                gg = g0 * NBUF + b
                h = b % 2
                nh = (b + 1) % 2
                nb = (b + 1) % NBUF
                lseq = g0 * (NBUF // 2) + b // 2
                seq = s0 + lseq
                nlseq = lseq + (1 if h == 1 else 0)

                @pl.when(gg >= NBUF - 1)
                def _():
                    drain_scatter(nh, nb)

                @pl.when(gg + 1 < G)
                def _():
                    fire_gather(nlseq, nh, nb)

                drain_gather(h, b)
                off, cnt = SUB[h]
                pltpu.async_copy(
                    rows[b].at[pl.ds(0, cnt)],
                    out_hbm.at[seq, pl.ds(off, cnt), pl.ds(0, D_MODEL)],
                    ssem[b],
                )
            return carry

        lax.fori_loop(0, G // NBUF, outer, 0)

        for b in range(1, NBUF):
            drain_scatter(b % 2, b)

    return emb_kernel


def kernel(x, W):
    B0, S = x.shape
    V = W.shape[0]
    w_packed = _make_format_kernel(V)(W.T)
    vt = (V // DPAD) * DPAD
    tail = (W[vt:] * jnp.float32(SCALE)).reshape(-1, DPAD)
    w_packed = lax.dynamic_update_slice(w_packed, tail, (vt // 2, 0))
    w_lin = w_packed.reshape(V, D_MODEL)
    out = _make_lookup_kernel(B0, S, V)(x.astype(jnp.int32), w_lin)
    return out[:, :, :D_MODEL]


# speedup vs baseline: 1.1346x; 1.1346x over previous
import functools
import math

import jax
import jax.numpy as jnp
from jax import lax
from jax.experimental import pallas as pl
from jax.experimental.pallas import tpu as pltpu
from jax.experimental.pallas import tpu_sc as plsc

D_MODEL = 64
DPAD = 128
SCALE = math.sqrt(D_MODEL)

NC, NS, L = 2, 16, 16
NW = NC * NS

NBUF = 4
SUB = ((0, 104), (104, 96))


def _make_format_kernel(V):
    full_chunks = V // DPAD
    tail_cols = V - full_chunks * DPAD
    iters = ((full_chunks + NW - 1) // NW + 1) // 2 * 2
    mesh = plsc.VectorSubcoreMesh(core_axis_name="c", subcore_axis_name="s")

    scratch = [pltpu.VMEM((D_MODEL, DPAD), jnp.float32) for _ in range(4)]
    scratch += [pltpu.SemaphoreType.DMA for _ in range(4)]

    @functools.partial(
        pl.kernel,
        out_type=jax.ShapeDtypeStruct((V // 2, DPAD), jnp.float32),
        mesh=mesh,
        scratch_types=scratch,
        compiler_params=pltpu.CompilerParams(
            use_tc_tiling_on_sc=True, needs_layout_passes=False
        ),
    )
    def fmt_kernel(wt_hbm, out_hbm, *bufs_and_sems):
        ins = bufs_and_sems[0:2]
        outs = bufs_and_sems[2:4]
        rsem = bufs_and_sems[4:6]
        wsem = bufs_and_sems[6:8]

        wid = lax.axis_index("s") * NC + lax.axis_index("c")
        lanes = lax.iota(jnp.int32, L)

        def chunk_of(i):
            c = wid + i * NW
            return jnp.minimum(c, full_chunks - 1)

        def fire_read(i, b):
            c = chunk_of(i)
            pltpu.async_copy(
                wt_hbm.at[pl.ds(0, D_MODEL), pl.ds(c * DPAD, DPAD)],
                ins[b],
                rsem[b],
            )

        def wait_read(b):
            pltpu.make_async_copy(
                wt_hbm.at[pl.ds(0, D_MODEL), pl.ds(0, DPAD)], ins[b], rsem[b]
            ).wait()

        def wait_write(b):
            pltpu.make_async_copy(outs[b], out_hbm.at[pl.ds(0, D_MODEL)], wsem[b]).wait()

        rows16s = [lanes + (16 * k) % D_MODEL for k in range(DPAD // L)]
        zeros16 = jnp.full((L,), 0, jnp.int32)

        def transpose_scale(b, nrows):
            src = ins[b]
            dst = outs[b]

            @plsc.parallel_loop(0, nrows, unroll=16)
            def _(j):
                c0 = zeros16 + 2 * j
                c1 = c0 + 1
                for k in range(DPAD // L):
                    cols16 = c0 if k < D_MODEL // L else c1
                    v = plsc.load_gather(src, [rows16s[k], cols16])
                    dst[j, pl.ds(k * L, L)] = v * SCALE

        fire_read(0, 0)

        def body(i0, carry):
            for b in range(2):
                i = i0 * 2 + b
                nb = 1 - b

                @pl.when(i + 1 < iters)
                def _():
                    fire_read(i + 1, nb)

                wait_read(b)

                @pl.when(i >= 2)
                def _():
                    wait_write(b)

                transpose_scale(b, D_MODEL)
                c = chunk_of(i)
                pltpu.async_copy(
                    outs[b], out_hbm.at[pl.ds(c * D_MODEL, D_MODEL)], wsem[b]
                )
            return carry

        lax.fori_loop(0, iters // 2, body, 0)
        wait_write(0)
        wait_write(1)

    return fmt_kernel


def _make_lookup_kernel(B0, S, V):
    seq_per_w = B0 // NW
    G = seq_per_w * len(SUB)
    assert G % NBUF == 0
    cmax = max(c for _, c in SUB)
    mesh = plsc.VectorSubcoreMesh(core_axis_name="c", subcore_axis_name="s")

    scratch = [pltpu.VMEM((seq_per_w, S), jnp.int32)]
    scratch += [pltpu.VMEM((cmax, D_MODEL), jnp.float32) for _ in range(NBUF)]
    scratch += [pltpu.SemaphoreType.DMA for _ in range(2 * NBUF)]

    @functools.partial(
        pl.kernel,
        out_type=jax.ShapeDtypeStruct((B0, S, DPAD), jnp.float32),
        mesh=mesh,
        scratch_types=scratch,
        compiler_params=pltpu.CompilerParams(use_tc_tiling_on_sc=False),
    )
    def emb_kernel(x_hbm, w_hbm, out_hbm, idx_v, *bufs_and_sems):
        rows = bufs_and_sems[:NBUF]
        gsem = bufs_and_sems[NBUF:2 * NBUF]
        ssem = bufs_and_sems[2 * NBUF:]

        wid = lax.axis_index("s") * NC + lax.axis_index("c")
        s0 = wid * seq_per_w

        pltpu.sync_copy(x_hbm.at[pl.ds(s0, seq_per_w)], idx_v)

        def fire_gather(lseq, h, b):
            off, cnt = SUB[h]
            pltpu.async_copy(
                w_hbm.at[idx_v.at[lseq, pl.ds(off, cnt)]],
                rows[b].at[pl.ds(0, cnt)],
                gsem[b],
            )

        def drain_gather(h, b):
            off, cnt = SUB[h]
            pltpu.make_async_copy(
                w_hbm.at[idx_v.at[0, pl.ds(off, cnt)]],
                rows[b].at[pl.ds(0, cnt)],
                gsem[b],
            ).wait()

        def drain_scatter(h, b):
            off, cnt = SUB[h]
            pltpu.make_async_copy(
                rows[b].at[pl.ds(0, cnt)],
                out_hbm.at[0, pl.ds(off, cnt), pl.ds(0, D_MODEL)],
                ssem[b],
            ).wait()

        fire_gather(0, 0, 0)

        def outer(g0, carry):
            for b in range(NBUF):
                gg = g0 * NBUF + b
                h = b % 2
                nh = (b + 1) % 2
                nb = (b + 1) % NBUF
                lseq = g0 * (NBUF // 2) + b // 2
                seq = s0 + lseq
                nlseq = lseq + (1 if h == 1 else 0)

                @pl.when(gg >= NBUF - 1)
                def _():
                    drain_scatter(nh, nb)

                @pl.when(gg + 1 < G)
                def _():
                    fire_gather(nlseq, nh, nb)

                drain_gather(h, b)
                off, cnt = SUB[h]
                pltpu.async_copy(
                    rows[b].at[pl.ds(0, cnt)],
                    out_hbm.at[seq, pl.ds(off, cnt), pl.ds(0, D_MODEL)],
                    ssem[b],
                )
            return carry

        lax.fori_loop(0, G // NBUF, outer, 0)

        for b in range(1, NBUF):
            drain_scatter(b % 2, b)

    return emb_kernel


def kernel(x, W):
    B0, S = x.shape
    V = W.shape[0]
    w_pad = jnp.pad(W * jnp.float32(SCALE), ((0, 0), (0, DPAD - D_MODEL)))
    w_lin = w_pad.reshape(2 * V, D_MODEL)
    x2 = x.astype(jnp.int32) * 2
    out = _make_lookup_kernel(B0, S, V)(x2, w_lin)
    return out[:, :, :D_MODEL]
